# Initial kernel scaffold; baseline (speedup 1.0000x reference)
import jax
import jax.numpy as jnp
from jax.experimental import pallas as pl


def kernel(x, pos_table):
    raise NotImplementedError("write your pallas kernel here")


# speedup vs baseline: 1.4896x; 1.4896x over previous
import jax
import jax.numpy as jnp
from jax.experimental import pallas as pl

BT = 512


def _add_kernel(x_ref, pos_ref, out_ref):
    out_ref[0, :, :] = x_ref[0, :, :] + pos_ref[:, :]


def kernel(x, pos_table):
    B, T, C = x.shape
    grid = (T // BT, B)
    return pl.pallas_call(
        _add_kernel,
        grid=grid,
        in_specs=[
            pl.BlockSpec((1, BT, C), lambda t, b: (b, t, 0)),
            pl.BlockSpec((BT, C), lambda t, b: (t, 0)),
        ],
        out_specs=pl.BlockSpec((1, BT, C), lambda t, b: (b, t, 0)),
        out_shape=jax.ShapeDtypeStruct((B, T, C), x.dtype),
    )(x, pos_table)


# speedup vs baseline: 1.6642x; 1.1172x over previous
import jax
import jax.numpy as jnp
from jax.experimental import pallas as pl

BT = 1024


def _add_kernel(x_ref, pos_ref, out_ref):
    out_ref[0, :, :] = x_ref[0, :, :] + pos_ref[:, :]


def kernel(x, pos_table):
    B, T, C = x.shape
    grid = (T // BT, B)
    return pl.pallas_call(
        _add_kernel,
        grid=grid,
        in_specs=[
            pl.BlockSpec((1, BT, C), lambda t, b: (b, t, 0)),
            pl.BlockSpec((BT, C), lambda t, b: (t, 0)),
        ],
        out_specs=pl.BlockSpec((1, BT, C), lambda t, b: (b, t, 0)),
        out_shape=jax.ShapeDtypeStruct((B, T, C), x.dtype),
    )(x, pos_table)


# speedup vs baseline: 1.7358x; 1.0430x over previous
import jax
import jax.numpy as jnp
from jax.experimental import pallas as pl

BT = 2048


def _add_kernel(x_ref, pos_ref, out_ref):
    out_ref[0, :, :] = x_ref[0, :, :] + pos_ref[:, :]


def kernel(x, pos_table):
    B, T, C = x.shape
    grid = (T // BT, B)
    return pl.pallas_call(
        _add_kernel,
        grid=grid,
        in_specs=[
            pl.BlockSpec((1, BT, C), lambda t, b: (b, t, 0)),
            pl.BlockSpec((BT, C), lambda t, b: (t, 0)),
        ],
        out_specs=pl.BlockSpec((1, BT, C), lambda t, b: (b, t, 0)),
        out_shape=jax.ShapeDtypeStruct((B, T, C), x.dtype),
    )(x, pos_table)


# speedup vs baseline: 1.7359x; 1.0000x over previous
import jax
import jax.numpy as jnp
from jax.experimental import pallas as pl
from jax.experimental.pallas import tpu as pltpu

BT = 2048


def _add_kernel(x_ref, pos_ref, out_ref):
    out_ref[0, :, :] = x_ref[0, :, :] + pos_ref[:, :]


def kernel(x, pos_table):
    B, T, C = x.shape
    grid = (T // BT, B)
    return pl.pallas_call(
        _add_kernel,
        grid=grid,
        in_specs=[
            pl.BlockSpec((1, BT, C), lambda t, b: (b, t, 0)),
            pl.BlockSpec((BT, C), lambda t, b: (t, 0)),
        ],
        out_specs=pl.BlockSpec((1, BT, C), lambda t, b: (b, t, 0)),
        out_shape=jax.ShapeDtypeStruct((B, T, C), x.dtype),
        compiler_params=pltpu.CompilerParams(
            dimension_semantics=("parallel", "arbitrary"),
        ),
    )(x, pos_table)
